# baseline (speedup 1.0000x reference)
import jax
import jax.numpy as jnp
from jax import lax
from jax.experimental import pallas as pl
from jax.experimental.pallas import tpu as pltpu
from jax.experimental.pallas import tpu_sc as plsc

N_FIELDS = 26
VOCAB = 100000
DIM = 32
B = 16384
NC, NS, L = 2, 16, 16
NW = NC * NS
BPW = B // NW


def _gather_body(xT_hbm, W_hbm, out_hbm, idx_v, rows_v, sem):
    wid = lax.axis_index("s") * NC + lax.axis_index("c")
    base = wid * BPW
    for j in range(N_FIELDS):
        pltpu.sync_copy(xT_hbm.at[j, pl.ds(base, BPW)], idx_v)

        def _add_off(i, carry):
            sl = pl.ds(i * L, L)
            idx_v[sl] = idx_v[sl] + (j * VOCAB)
            return carry

        lax.fori_loop(0, BPW // L, _add_off, 0)
        pltpu.async_copy(W_hbm.at[idx_v], rows_v, sem).wait()
        pltpu.sync_copy(rows_v, out_hbm.at[pl.ds(j * B + base, BPW)])


def kernel(x, W):
    xT = x.T
    Wf = W.reshape(N_FIELDS * VOCAB, DIM)
    mesh = plsc.VectorSubcoreMesh(
        core_axis_name="c", subcore_axis_name="s",
        num_cores=NC, num_subcores=NS,
    )
    out = pl.kernel(
        _gather_body,
        out_type=jax.ShapeDtypeStruct((N_FIELDS * B, DIM), jnp.float32),
        mesh=mesh,
        scratch_types=[
            pltpu.VMEM((BPW,), jnp.int32),
            pltpu.VMEM((BPW, DIM), jnp.float32),
            pltpu.SemaphoreType.DMA,
        ],
        compiler_params=pltpu.CompilerParams(use_tc_tiling_on_sc=False),
    )(xT, Wf)
    return out.reshape(N_FIELDS, B, DIM)


# speedup vs baseline: 1.0215x; 1.0215x over previous
import jax
import jax.numpy as jnp
from jax import lax
from jax.experimental import pallas as pl
from jax.experimental.pallas import tpu as pltpu
from jax.experimental.pallas import tpu_sc as plsc

N_FIELDS = 26
VOCAB = 100000
DIM = 32
B = 16384
NC, NS, L = 2, 16, 16
NW = NC * NS
BPW = B // NW
NBUF = 3


def _gather_body(xT_hbm, W_hbm, out_hbm, idx_v, rows_v, gsem, osem):
    wid = lax.axis_index("s") * NC + lax.axis_index("c")
    base = wid * BPW

    def load_idx(j, s):
        pltpu.sync_copy(xT_hbm.at[j, pl.ds(base, BPW)], idx_v.at[s])

        def _add(i, carry):
            sl = pl.ds(i * L, L)
            idx_v[s, sl] = idx_v[s, sl] + (j * VOCAB)
            return carry

        lax.fori_loop(0, BPW // L, _add, 0)

    def gather(j, s):
        return pltpu.make_async_copy(W_hbm.at[idx_v.at[s]], rows_v.at[s],
                                     gsem.at[s])

    def writeback(j, s):
        return pltpu.make_async_copy(rows_v.at[s],
                                     out_hbm.at[pl.ds(j * B + base, BPW)],
                                     osem.at[s])

    for j in range(N_FIELDS + 1):
        if j < N_FIELDS:
            s = j % NBUF
            if j >= NBUF:
                writeback(j - NBUF, s).wait()
            load_idx(j, s)
            gather(j, s).start()
        if 1 <= j:
            s1 = (j - 1) % NBUF
            gather(j - 1, s1).wait()
            writeback(j - 1, s1).start()
    for j in range(N_FIELDS - NBUF + 1, N_FIELDS):
        writeback(j, j % NBUF).wait()


def kernel(x, W):
    xT = x.T
    Wf = W.reshape(N_FIELDS * VOCAB, DIM)
    mesh = plsc.VectorSubcoreMesh(
        core_axis_name="c", subcore_axis_name="s",
        num_cores=NC, num_subcores=NS,
    )
    out = pl.kernel(
        _gather_body,
        out_type=jax.ShapeDtypeStruct((N_FIELDS * B, DIM), jnp.float32),
        mesh=mesh,
        scratch_types=[
            pltpu.VMEM((NBUF, BPW), jnp.int32),
            pltpu.VMEM((NBUF, BPW, DIM), jnp.float32),
            pltpu.SemaphoreType.DMA((NBUF,)),
            pltpu.SemaphoreType.DMA((NBUF,)),
        ],
        compiler_params=pltpu.CompilerParams(use_tc_tiling_on_sc=False),
    )(xT, Wf)
    return out.reshape(N_FIELDS, B, DIM)


# speedup vs baseline: 1.0217x; 1.0001x over previous
import jax
import jax.numpy as jnp
from jax import lax
from jax.experimental import pallas as pl
from jax.experimental.pallas import tpu as pltpu
from jax.experimental.pallas import tpu_sc as plsc

N_FIELDS = 26
VOCAB = 100000
DIM = 32
B = 16384
NC, NS, L = 2, 16, 16
NW = NC * NS
BPW = B // NW
NBUF = 3


def _gather_body(x_hbm, W_hbm, out_hbm, xblk_v, idx_v, rows_v, gsem, osem):
    wid = lax.axis_index("s") * NC + lax.axis_index("c")
    base = wid * BPW

    pltpu.sync_copy(x_hbm.at[pl.ds(base * N_FIELDS, BPW * N_FIELDS)], xblk_v)
    lane = lax.broadcasted_iota(jnp.int32, (L,), 0)

    def load_idx(j, s):
        col = lane * N_FIELDS + j

        def _mk(i, carry):
            g = plsc.load_gather(xblk_v, [col + i * (L * N_FIELDS)])
            idx_v[s, pl.ds(i * L, L)] = g
            return carry

        lax.fori_loop(0, BPW // L, _mk, 0)

    def gather(j, s):
        return pltpu.make_async_copy(W_hbm.at[j].at[idx_v.at[s]],
                                     rows_v.at[s], gsem.at[s])

    def writeback(j, s):
        return pltpu.make_async_copy(rows_v.at[s],
                                     out_hbm.at[j, pl.ds(base, BPW)],
                                     osem.at[s])

    for j in range(N_FIELDS + 1):
        if j < N_FIELDS:
            s = j % NBUF
            if j >= NBUF:
                writeback(j - NBUF, s).wait()
            load_idx(j, s)
            gather(j, s).start()
        if 1 <= j:
            s1 = (j - 1) % NBUF
            gather(j - 1, s1).wait()
            writeback(j - 1, s1).start()
    for j in range(N_FIELDS - NBUF + 1, N_FIELDS):
        writeback(j, j % NBUF).wait()


def kernel(x, W):
    mesh = plsc.VectorSubcoreMesh(
        core_axis_name="c", subcore_axis_name="s",
        num_cores=NC, num_subcores=NS,
    )
    return pl.kernel(
        _gather_body,
        out_type=jax.ShapeDtypeStruct((N_FIELDS, B, DIM), jnp.float32),
        mesh=mesh,
        scratch_types=[
            pltpu.VMEM((BPW * N_FIELDS,), jnp.int32),
            pltpu.VMEM((NBUF, BPW), jnp.int32),
            pltpu.VMEM((NBUF, BPW, DIM), jnp.float32),
            pltpu.SemaphoreType.DMA((NBUF,)),
            pltpu.SemaphoreType.DMA((NBUF,)),
        ],
        compiler_params=pltpu.CompilerParams(use_tc_tiling_on_sc=False,
                                             needs_layout_passes=False),
    )(x.reshape(B * N_FIELDS), W)
